# Initial kernel scaffold; baseline (speedup 1.0000x reference)
import jax
import jax.numpy as jnp
from jax.experimental import pallas as pl


def kernel(user_emb, item_emb, edge_index_u2i, edge_index_i2u, edge_label_index):
    raise NotImplementedError("write your pallas kernel here")


# speedup vs baseline: 3.4562x; 3.4562x over previous
import functools

import jax
import jax.numpy as jnp
from jax import lax
from jax.experimental import pallas as pl
from jax.experimental.pallas import tpu as pltpu
from jax.experimental.pallas import tpu_sc as plsc

_NU = 50000
_NI = 50000
_H = 32
_DEPTH = 2
_B = 16384

_info = plsc.get_sparse_core_info()
_NC, _NS = _info.num_cores, _info.num_subcores
_NW = _NC * _NS


def _decoder_sc(out_user, out_item, idx_u, idx_i):
    b_per_w = _B // _NW
    mesh = plsc.VectorSubcoreMesh(core_axis_name="c", subcore_axis_name="s")

    @functools.partial(
        pl.kernel,
        mesh=mesh,
        compiler_params=pltpu.CompilerParams(
            needs_layout_passes=False, use_tc_tiling_on_sc=False
        ),
        out_type=jax.ShapeDtypeStruct((_B,), jnp.float32),
        scratch_types=[
            pltpu.VMEM((b_per_w,), jnp.int32),
            pltpu.VMEM((b_per_w,), jnp.int32),
            pltpu.VMEM((b_per_w, _H), jnp.float32),
            pltpu.VMEM((b_per_w, _H), jnp.float32),
            pltpu.VMEM((b_per_w,), jnp.float32),
            pltpu.SemaphoreType.DMA,
        ],
    )
    def dec(u_hbm, i_hbm, iu_hbm, ii_hbm, out_hbm, iu_v, ii_v, ru_v, ri_v, o_v, sem):
        wid = lax.axis_index("s") * _NC + lax.axis_index("c")
        base = wid * b_per_w
        pltpu.sync_copy(iu_hbm.at[pl.ds(base, b_per_w)], iu_v)
        pltpu.sync_copy(ii_hbm.at[pl.ds(base, b_per_w)], ii_v)
        pltpu.async_copy(u_hbm.at[iu_v], ru_v, sem).wait()
        pltpu.async_copy(i_hbm.at[ii_v], ri_v, sem).wait()

        def body(g, carry):
            lanes = lax.iota(jnp.int32, 16)
            dots = jnp.zeros((16,), jnp.float32)
            for l in range(16):
                j = g * 16 + l
                a = ru_v[j, pl.ds(0, 16)] * ri_v[j, pl.ds(0, 16)]
                b = ru_v[j, pl.ds(16, 16)] * ri_v[j, pl.ds(16, 16)]
                s = jnp.sum(a + b)
                dots = jnp.where(lanes == l, s, dots)
            o_v[pl.ds(g * 16, 16)] = dots
            return carry

        lax.fori_loop(0, b_per_w // 16, body, 0)
        pltpu.sync_copy(o_v, out_hbm.at[pl.ds(base, b_per_w)])

    return dec(out_user, out_item, idx_u, idx_i)


def kernel(user_emb, item_emb, edge_index_u2i, edge_index_i2u, edge_label_index):
    u_all = jnp.concatenate([edge_index_u2i[0], edge_index_i2u[1]])
    i_all = jnp.concatenate([edge_index_u2i[1], edge_index_i2u[0]])
    key = u_all.astype(jnp.uint32) * jnp.uint32(_NI) + i_all.astype(jnp.uint32)
    skey = lax.sort(key)
    first = jnp.concatenate(
        [jnp.ones((1,), bool), skey[1:] != skey[:-1]]
    )
    su = (skey // jnp.uint32(_NI)).astype(jnp.int32)
    si = (skey % jnp.uint32(_NI)).astype(jnp.int32)
    m = first.astype(jnp.float32)
    deg_u = jax.ops.segment_sum(m, su, num_segments=_NU)
    deg_i = jax.ops.segment_sum(m, si, num_segments=_NI)
    dinv_u = jnp.where(deg_u > 0, jnp.where(deg_u > 0, deg_u, 1.0) ** -0.5, 0.0)
    dinv_i = jnp.where(deg_i > 0, jnp.where(deg_i > 0, deg_i, 1.0) ** -0.5, 0.0)

    alpha = 1.0 / (_DEPTH + 1)
    xu, xi = user_emb, item_emb
    ou = xu * alpha
    oi = xi * alpha
    for _ in range(_DEPTH):
        gu = (xu * dinv_u[:, None])[su] * m[:, None]
        gi = (xi * dinv_i[:, None])[si] * m[:, None]
        xu = jax.ops.segment_sum(gi, su, num_segments=_NU) * dinv_u[:, None]
        xi = jax.ops.segment_sum(gu, si, num_segments=_NI) * dinv_i[:, None]
        ou = ou + xu * alpha
        oi = oi + xi * alpha

    return _decoder_sc(ou, oi, edge_label_index[0], edge_label_index[1])


# speedup vs baseline: 15.0590x; 4.3571x over previous
import functools

import jax
import jax.numpy as jnp
from jax import lax
from jax.experimental import pallas as pl
from jax.experimental.pallas import tpu as pltpu
from jax.experimental.pallas import tpu_sc as plsc

_NU = 50000
_NI = 50000
_H = 32
_DEPTH = 2
_B = 16384

_info = plsc.get_sparse_core_info()
_NC, _NS = _info.num_cores, _info.num_subcores
_NW = _NC * _NS


def _decoder_sc(out_user, out_item, idx_u, idx_i):
    b_per_w = _B // _NW
    mesh = plsc.VectorSubcoreMesh(core_axis_name="c", subcore_axis_name="s")

    @functools.partial(
        pl.kernel,
        mesh=mesh,
        compiler_params=pltpu.CompilerParams(
            needs_layout_passes=False, use_tc_tiling_on_sc=False
        ),
        out_type=jax.ShapeDtypeStruct((_B,), jnp.float32),
        scratch_types=[
            pltpu.VMEM((b_per_w,), jnp.int32),
            pltpu.VMEM((b_per_w,), jnp.int32),
            pltpu.VMEM((b_per_w, _H), jnp.float32),
            pltpu.VMEM((b_per_w, _H), jnp.float32),
            pltpu.VMEM((b_per_w,), jnp.float32),
            pltpu.SemaphoreType.DMA,
        ],
    )
    def dec(u_hbm, i_hbm, iu_hbm, ii_hbm, out_hbm, iu_v, ii_v, ru_v, ri_v, o_v, sem):
        wid = lax.axis_index("s") * _NC + lax.axis_index("c")
        base = wid * b_per_w
        pltpu.sync_copy(iu_hbm.at[pl.ds(base, b_per_w)], iu_v)
        pltpu.sync_copy(ii_hbm.at[pl.ds(base, b_per_w)], ii_v)
        pltpu.async_copy(u_hbm.at[iu_v], ru_v, sem).wait()
        pltpu.async_copy(i_hbm.at[ii_v], ri_v, sem).wait()

        def body(g, carry):
            lanes = lax.iota(jnp.int32, 16)
            dots = jnp.zeros((16,), jnp.float32)
            for l in range(16):
                j = g * 16 + l
                a = ru_v[j, pl.ds(0, 16)] * ri_v[j, pl.ds(0, 16)]
                b = ru_v[j, pl.ds(16, 16)] * ri_v[j, pl.ds(16, 16)]
                s = jnp.sum(a + b)
                dots = jnp.where(lanes == l, s, dots)
            o_v[pl.ds(g * 16, 16)] = dots
            return carry

        lax.fori_loop(0, b_per_w // 16, body, 0)
        pltpu.sync_copy(o_v, out_hbm.at[pl.ds(base, b_per_w)])

    return dec(out_user, out_item, idx_u, idx_i)


_NPAD = 48
_NT = _NU + _NPAD
_CH = 512
_EPT = 102400
_EPAD = _EPT * _NS


def _propagate_sc(xsu, xsi, su_eff, si_eff, zeros_tab):
    mesh = plsc.VectorSubcoreMesh(core_axis_name="c", subcore_axis_name="s")

    @functools.partial(
        pl.kernel,
        mesh=mesh,
        compiler_params=pltpu.CompilerParams(
            needs_layout_passes=False, use_tc_tiling_on_sc=False
        ),
        out_type=(
            jax.ShapeDtypeStruct((_NU, _H), jnp.float32),
            jax.ShapeDtypeStruct((_NI, _H), jnp.float32),
        ),
        scratch_types=[
            pltpu.VMEM((_CH,), jnp.int32),
            pltpu.VMEM((_CH,), jnp.int32),
            pltpu.VMEM((_CH, _H), jnp.float32),
            pltpu.VMEM_SHARED((_NT, _H), jnp.float32),
            pltpu.SemaphoreType.DMA,
        ],
    )
    def prop(xu_hbm, xi_hbm, su_hbm, si_hbm, z_hbm, ou_hbm, oi_hbm,
             gi_v, si_v, rows_v, acc_sh, sem):
        c = lax.axis_index("c")
        t = lax.axis_index("s")
        rpt = _NT // _NS
        pltpu.sync_copy(z_hbm.at[pl.ds(t * rpt, rpt)], acc_sh.at[pl.ds(t * rpt, rpt)])
        plsc.subcore_barrier()

        def run(gtab_hbm, gidx_hbm, sidx_hbm):
            def body(g, carry):
                e0 = t * _EPT + g * _CH
                pltpu.sync_copy(gidx_hbm.at[pl.ds(e0, _CH)], gi_v)
                pltpu.sync_copy(sidx_hbm.at[pl.ds(e0, _CH)], si_v)
                pltpu.async_copy(gtab_hbm.at[gi_v], rows_v, sem).wait()
                pltpu.sync_copy(rows_v, acc_sh.at[si_v], add=True)
                return carry

            lax.fori_loop(0, _EPT // _CH, body, 0)

        @pl.when(c == 0)
        def _():
            run(xi_hbm, si_hbm, su_hbm)

        @pl.when(c == 1)
        def _():
            run(xu_hbm, su_hbm, si_hbm)

        plsc.subcore_barrier()
        wpt = _NU // _NS

        @pl.when(c == 0)
        def _():
            pltpu.sync_copy(
                acc_sh.at[pl.ds(t * wpt, wpt)], ou_hbm.at[pl.ds(t * wpt, wpt)]
            )

        @pl.when(c == 1)
        def _():
            pltpu.sync_copy(
                acc_sh.at[pl.ds(t * wpt, wpt)], oi_hbm.at[pl.ds(t * wpt, wpt)]
            )

    return prop(xsu, xsi, su_eff, si_eff, zeros_tab)


def kernel(user_emb, item_emb, edge_index_u2i, edge_index_i2u, edge_label_index):
    u_all = jnp.concatenate([edge_index_u2i[0], edge_index_i2u[1]])
    i_all = jnp.concatenate([edge_index_u2i[1], edge_index_i2u[0]])
    key = u_all.astype(jnp.uint32) * jnp.uint32(_NI) + i_all.astype(jnp.uint32)
    skey = lax.sort(key)
    first = jnp.concatenate(
        [jnp.ones((1,), bool), skey[1:] != skey[:-1]]
    )
    su = (skey // jnp.uint32(_NI)).astype(jnp.int32)
    si = (skey % jnp.uint32(_NI)).astype(jnp.int32)
    m = first.astype(jnp.float32)
    deg_u = jax.ops.segment_sum(m, su, num_segments=_NU)
    deg_i = jax.ops.segment_sum(m, si, num_segments=_NI)
    dinv_u = jnp.where(deg_u > 0, jnp.where(deg_u > 0, deg_u, 1.0) ** -0.5, 0.0)
    dinv_i = jnp.where(deg_i > 0, jnp.where(deg_i > 0, deg_i, 1.0) ** -0.5, 0.0)

    e_tot = su.shape[0]
    dummy = (jnp.arange(e_tot, dtype=jnp.int32) % _NPAD) + _NU
    su_eff = jnp.where(first, su, dummy)
    si_eff = jnp.where(first, si, dummy)
    pad_idx = (jnp.arange(_EPAD - e_tot, dtype=jnp.int32) % _NPAD) + _NU
    su_eff = jnp.concatenate([su_eff, pad_idx])
    si_eff = jnp.concatenate([si_eff, pad_idx])
    zeros_tab = jnp.zeros((_NT, _H), jnp.float32)
    zpad = jnp.zeros((_NPAD, _H), jnp.float32)

    alpha = 1.0 / (_DEPTH + 1)
    xu, xi = user_emb, item_emb
    ou = xu * alpha
    oi = xi * alpha
    for _ in range(_DEPTH):
        xsu = jnp.concatenate([xu * dinv_u[:, None], zpad])
        xsi = jnp.concatenate([xi * dinv_i[:, None], zpad])
        au, ai = _propagate_sc(xsu, xsi, su_eff, si_eff, zeros_tab)
        xu = au * dinv_u[:, None]
        xi = ai * dinv_i[:, None]
        ou = ou + xu * alpha
        oi = oi + xi * alpha

    return _decoder_sc(ou, oi, edge_label_index[0], edge_label_index[1])


# speedup vs baseline: 15.0883x; 1.0019x over previous
import functools

import jax
import jax.numpy as jnp
from jax import lax
from jax.experimental import pallas as pl
from jax.experimental.pallas import tpu as pltpu
from jax.experimental.pallas import tpu_sc as plsc
from jax.experimental.compute_on import compute_on

_NU = 50000
_NI = 50000
_H = 32
_DEPTH = 2
_B = 16384

_info = plsc.get_sparse_core_info()
_NC, _NS = _info.num_cores, _info.num_subcores
_NW = _NC * _NS


def _decoder_sc(out_user, out_item, idx_u, idx_i):
    b_per_w = _B // _NW
    mesh = plsc.VectorSubcoreMesh(core_axis_name="c", subcore_axis_name="s")

    @functools.partial(
        pl.kernel,
        mesh=mesh,
        compiler_params=pltpu.CompilerParams(
            needs_layout_passes=False, use_tc_tiling_on_sc=False
        ),
        out_type=jax.ShapeDtypeStruct((_B,), jnp.float32),
        scratch_types=[
            pltpu.VMEM((b_per_w,), jnp.int32),
            pltpu.VMEM((b_per_w,), jnp.int32),
            pltpu.VMEM((b_per_w, _H), jnp.float32),
            pltpu.VMEM((b_per_w, _H), jnp.float32),
            pltpu.VMEM((b_per_w,), jnp.float32),
            pltpu.SemaphoreType.DMA,
        ],
    )
    def dec(u_hbm, i_hbm, iu_hbm, ii_hbm, out_hbm, iu_v, ii_v, ru_v, ri_v, o_v, sem):
        wid = lax.axis_index("s") * _NC + lax.axis_index("c")
        base = wid * b_per_w
        pltpu.sync_copy(iu_hbm.at[pl.ds(base, b_per_w)], iu_v)
        pltpu.sync_copy(ii_hbm.at[pl.ds(base, b_per_w)], ii_v)
        pltpu.async_copy(u_hbm.at[iu_v], ru_v, sem).wait()
        pltpu.async_copy(i_hbm.at[ii_v], ri_v, sem).wait()

        def body(g, carry):
            lanes = lax.iota(jnp.int32, 16)
            dots = jnp.zeros((16,), jnp.float32)
            for l in range(16):
                j = g * 16 + l
                a = ru_v[j, pl.ds(0, 16)] * ri_v[j, pl.ds(0, 16)]
                b = ru_v[j, pl.ds(16, 16)] * ri_v[j, pl.ds(16, 16)]
                s = jnp.sum(a + b)
                dots = jnp.where(lanes == l, s, dots)
            o_v[pl.ds(g * 16, 16)] = dots
            return carry

        lax.fori_loop(0, b_per_w // 16, body, 0)
        pltpu.sync_copy(o_v, out_hbm.at[pl.ds(base, b_per_w)])

    return dec(out_user, out_item, idx_u, idx_i)


_NPAD = 48
_NT = _NU + _NPAD
_CH = 512
_EPT = 102400
_EPAD = _EPT * _NS


def _propagate_sc(xsu, xsi, su_eff, si_eff, zeros_tab):
    mesh = plsc.VectorSubcoreMesh(core_axis_name="c", subcore_axis_name="s")

    @functools.partial(
        pl.kernel,
        mesh=mesh,
        compiler_params=pltpu.CompilerParams(
            needs_layout_passes=False, use_tc_tiling_on_sc=False
        ),
        out_type=(
            jax.ShapeDtypeStruct((_NU, _H), jnp.float32),
            jax.ShapeDtypeStruct((_NI, _H), jnp.float32),
        ),
        scratch_types=[
            pltpu.VMEM((_CH,), jnp.int32),
            pltpu.VMEM((_CH,), jnp.int32),
            pltpu.VMEM((_CH, _H), jnp.float32),
            pltpu.VMEM_SHARED((_NT, _H), jnp.float32),
            pltpu.SemaphoreType.DMA,
        ],
    )
    def prop(xu_hbm, xi_hbm, su_hbm, si_hbm, z_hbm, ou_hbm, oi_hbm,
             gi_v, si_v, rows_v, acc_sh, sem):
        c = lax.axis_index("c")
        t = lax.axis_index("s")
        rpt = _NT // _NS
        pltpu.sync_copy(z_hbm.at[pl.ds(t * rpt, rpt)], acc_sh.at[pl.ds(t * rpt, rpt)])
        plsc.subcore_barrier()

        def run(gtab_hbm, gidx_hbm, sidx_hbm):
            def body(g, carry):
                e0 = t * _EPT + g * _CH
                pltpu.sync_copy(gidx_hbm.at[pl.ds(e0, _CH)], gi_v)
                pltpu.sync_copy(sidx_hbm.at[pl.ds(e0, _CH)], si_v)
                pltpu.async_copy(gtab_hbm.at[gi_v], rows_v, sem).wait()
                pltpu.sync_copy(rows_v, acc_sh.at[si_v], add=True)
                return carry

            lax.fori_loop(0, _EPT // _CH, body, 0)

        @pl.when(c == 0)
        def _():
            run(xi_hbm, si_hbm, su_hbm)

        @pl.when(c == 1)
        def _():
            run(xu_hbm, su_hbm, si_hbm)

        plsc.subcore_barrier()
        wpt = _NU // _NS

        @pl.when(c == 0)
        def _():
            pltpu.sync_copy(
                acc_sh.at[pl.ds(t * wpt, wpt)], ou_hbm.at[pl.ds(t * wpt, wpt)]
            )

        @pl.when(c == 1)
        def _():
            pltpu.sync_copy(
                acc_sh.at[pl.ds(t * wpt, wpt)], oi_hbm.at[pl.ds(t * wpt, wpt)]
            )

    return prop(xsu, xsi, su_eff, si_eff, zeros_tab)


def kernel(user_emb, item_emb, edge_index_u2i, edge_index_i2u, edge_label_index):
    u_all = jnp.concatenate([edge_index_u2i[0], edge_index_i2u[1]])
    i_all = jnp.concatenate([edge_index_u2i[1], edge_index_i2u[0]])
    key = u_all.astype(jnp.uint32) * jnp.uint32(_NI) + i_all.astype(jnp.uint32)

    @compute_on("tpu_sparsecore")
    @jax.jit
    def _sc_sort(k):
        return lax.sort(k)

    skey = _sc_sort(key)
    first = jnp.concatenate(
        [jnp.ones((1,), bool), skey[1:] != skey[:-1]]
    )
    su = (skey // jnp.uint32(_NI)).astype(jnp.int32)
    si = (skey % jnp.uint32(_NI)).astype(jnp.int32)
    m = first.astype(jnp.float32)
    deg_u = jax.ops.segment_sum(m, su, num_segments=_NU)
    deg_i = jax.ops.segment_sum(m, si, num_segments=_NI)
    dinv_u = jnp.where(deg_u > 0, jnp.where(deg_u > 0, deg_u, 1.0) ** -0.5, 0.0)
    dinv_i = jnp.where(deg_i > 0, jnp.where(deg_i > 0, deg_i, 1.0) ** -0.5, 0.0)

    e_tot = su.shape[0]
    dummy = (jnp.arange(e_tot, dtype=jnp.int32) % _NPAD) + _NU
    su_eff = jnp.where(first, su, dummy)
    si_eff = jnp.where(first, si, dummy)
    pad_idx = (jnp.arange(_EPAD - e_tot, dtype=jnp.int32) % _NPAD) + _NU
    su_eff = jnp.concatenate([su_eff, pad_idx])
    si_eff = jnp.concatenate([si_eff, pad_idx])
    zeros_tab = jnp.zeros((_NT, _H), jnp.float32)
    zpad = jnp.zeros((_NPAD, _H), jnp.float32)

    alpha = 1.0 / (_DEPTH + 1)
    xu, xi = user_emb, item_emb
    ou = xu * alpha
    oi = xi * alpha
    for _ in range(_DEPTH):
        xsu = jnp.concatenate([xu * dinv_u[:, None], zpad])
        xsi = jnp.concatenate([xi * dinv_i[:, None], zpad])
        au, ai = _propagate_sc(xsu, xsi, su_eff, si_eff, zeros_tab)
        xu = au * dinv_u[:, None]
        xi = ai * dinv_i[:, None]
        ou = ou + xu * alpha
        oi = oi + xi * alpha

    return _decoder_sc(ou, oi, edge_label_index[0], edge_label_index[1])


# speedup vs baseline: 24.6725x; 1.6352x over previous
import functools

import jax
import jax.numpy as jnp
from jax import lax
from jax.experimental import pallas as pl
from jax.experimental.pallas import tpu as pltpu
from jax.experimental.pallas import tpu_sc as plsc
from jax.experimental.compute_on import compute_on

_NU = 50000
_NI = 50000
_H = 32
_DEPTH = 2
_B = 16384

_info = plsc.get_sparse_core_info()
_NC, _NS = _info.num_cores, _info.num_subcores
_NW = _NC * _NS


def _decoder_sc(out_user, out_item, idx_u, idx_i):
    b_per_w = _B // _NW
    mesh = plsc.VectorSubcoreMesh(core_axis_name="c", subcore_axis_name="s")

    @functools.partial(
        pl.kernel,
        mesh=mesh,
        compiler_params=pltpu.CompilerParams(
            needs_layout_passes=False, use_tc_tiling_on_sc=False
        ),
        out_type=jax.ShapeDtypeStruct((_B,), jnp.float32),
        scratch_types=[
            pltpu.VMEM((b_per_w,), jnp.int32),
            pltpu.VMEM((b_per_w,), jnp.int32),
            pltpu.VMEM((b_per_w, _H), jnp.float32),
            pltpu.VMEM((b_per_w, _H), jnp.float32),
            pltpu.VMEM((b_per_w,), jnp.float32),
            pltpu.SemaphoreType.DMA,
        ],
    )
    def dec(u_hbm, i_hbm, iu_hbm, ii_hbm, out_hbm, iu_v, ii_v, ru_v, ri_v, o_v, sem):
        wid = lax.axis_index("s") * _NC + lax.axis_index("c")
        base = wid * b_per_w
        pltpu.sync_copy(iu_hbm.at[pl.ds(base, b_per_w)], iu_v)
        pltpu.sync_copy(ii_hbm.at[pl.ds(base, b_per_w)], ii_v)
        pltpu.async_copy(u_hbm.at[iu_v], ru_v, sem).wait()
        pltpu.async_copy(i_hbm.at[ii_v], ri_v, sem).wait()

        def body(g, carry):
            lanes = lax.iota(jnp.int32, 16)
            dots = jnp.zeros((16,), jnp.float32)
            for l in range(16):
                j = g * 16 + l
                a = ru_v[j, pl.ds(0, 16)] * ri_v[j, pl.ds(0, 16)]
                b = ru_v[j, pl.ds(16, 16)] * ri_v[j, pl.ds(16, 16)]
                s = jnp.sum(a + b)
                dots = jnp.where(lanes == l, s, dots)
            o_v[pl.ds(g * 16, 16)] = dots
            return carry

        lax.fori_loop(0, b_per_w // 16, body, 0)
        pltpu.sync_copy(o_v, out_hbm.at[pl.ds(base, b_per_w)])

    return dec(out_user, out_item, idx_u, idx_i)


_NPAD = 176
_NT = _NU + _NPAD
_CH = 512
_EPT = 102400
_EPAD = _EPT * _NS


def _deg_dinv_sc(su_eff, si_eff, zeros_vec):
    mesh = plsc.VectorSubcoreMesh(core_axis_name="c", subcore_axis_name="s")
    rpt = _NT // _NS

    @functools.partial(
        pl.kernel,
        mesh=mesh,
        compiler_params=pltpu.CompilerParams(
            needs_layout_passes=False, use_tc_tiling_on_sc=False
        ),
        out_type=(
            jax.ShapeDtypeStruct((_NT,), jnp.float32),
            jax.ShapeDtypeStruct((_NT,), jnp.float32),
        ),
        scratch_types=[
            pltpu.VMEM((_CH,), jnp.int32),
            pltpu.VMEM((_CH,), jnp.float32),
            pltpu.VMEM_SHARED((_NT,), jnp.float32),
        ],
    )
    def degk(su_hbm, si_hbm, z_hbm, du_hbm, di_hbm, idx_v, ones_v, deg_sh):
        c = lax.axis_index("c")
        t = lax.axis_index("s")
        pltpu.sync_copy(z_hbm.at[pl.ds(t * rpt, rpt)], deg_sh.at[pl.ds(t * rpt, rpt)])
        for j in range(_CH // 16):
            ones_v[pl.ds(16 * j, 16)] = jnp.full((16,), 1.0, jnp.float32)
        plsc.subcore_barrier()

        def run(sidx_hbm):
            def body(g, carry):
                e0 = t * _EPT + g * _CH
                pltpu.sync_copy(sidx_hbm.at[pl.ds(e0, _CH)], idx_v)
                pltpu.sync_copy(ones_v, deg_sh.at[idx_v], add=True)
                return carry

            lax.fori_loop(0, _EPT // _CH, body, 0)

        @pl.when(c == 0)
        def _():
            run(su_hbm)

        @pl.when(c == 1)
        def _():
            run(si_hbm)

        plsc.subcore_barrier()

        @pl.when(c == 0)
        def _():
            pltpu.sync_copy(deg_sh.at[pl.ds(t * rpt, rpt)], du_hbm.at[pl.ds(t * rpt, rpt)])

        @pl.when(c == 1)
        def _():
            pltpu.sync_copy(deg_sh.at[pl.ds(t * rpt, rpt)], di_hbm.at[pl.ds(t * rpt, rpt)])

    return degk(su_eff, si_eff, zeros_vec)


def _propagate_sc(xsu, xsi, su_eff, si_eff, zeros_tab):
    mesh = plsc.VectorSubcoreMesh(core_axis_name="c", subcore_axis_name="s")

    @functools.partial(
        pl.kernel,
        mesh=mesh,
        compiler_params=pltpu.CompilerParams(
            needs_layout_passes=False, use_tc_tiling_on_sc=False
        ),
        out_type=(
            jax.ShapeDtypeStruct((_NU, _H), jnp.float32),
            jax.ShapeDtypeStruct((_NI, _H), jnp.float32),
        ),
        scratch_types=[
            pltpu.VMEM((_CH,), jnp.int32),
            pltpu.VMEM((_CH,), jnp.int32),
            pltpu.VMEM((_CH, _H), jnp.float32),
            pltpu.VMEM_SHARED((_NT, _H), jnp.float32),
            pltpu.SemaphoreType.DMA,
        ],
    )
    def prop(xu_hbm, xi_hbm, su_hbm, si_hbm, z_hbm, ou_hbm, oi_hbm,
             gi_v, si_v, rows_v, acc_sh, sem):
        c = lax.axis_index("c")
        t = lax.axis_index("s")
        rpt = _NT // _NS
        pltpu.sync_copy(z_hbm.at[pl.ds(t * rpt, rpt)], acc_sh.at[pl.ds(t * rpt, rpt)])
        plsc.subcore_barrier()

        def run(gtab_hbm, gidx_hbm, sidx_hbm):
            def body(g, carry):
                e0 = t * _EPT + g * _CH
                pltpu.sync_copy(gidx_hbm.at[pl.ds(e0, _CH)], gi_v)
                pltpu.sync_copy(sidx_hbm.at[pl.ds(e0, _CH)], si_v)
                pltpu.async_copy(gtab_hbm.at[gi_v], rows_v, sem).wait()
                pltpu.sync_copy(rows_v, acc_sh.at[si_v], add=True)
                return carry

            lax.fori_loop(0, _EPT // _CH, body, 0)

        @pl.when(c == 0)
        def _():
            run(xi_hbm, si_hbm, su_hbm)

        @pl.when(c == 1)
        def _():
            run(xu_hbm, su_hbm, si_hbm)

        plsc.subcore_barrier()
        wpt = _NU // _NS

        @pl.when(c == 0)
        def _():
            pltpu.sync_copy(
                acc_sh.at[pl.ds(t * wpt, wpt)], ou_hbm.at[pl.ds(t * wpt, wpt)]
            )

        @pl.when(c == 1)
        def _():
            pltpu.sync_copy(
                acc_sh.at[pl.ds(t * wpt, wpt)], oi_hbm.at[pl.ds(t * wpt, wpt)]
            )

    return prop(xsu, xsi, su_eff, si_eff, zeros_tab)


def kernel(user_emb, item_emb, edge_index_u2i, edge_index_i2u, edge_label_index):
    u_all = jnp.concatenate([edge_index_u2i[0], edge_index_i2u[1]])
    i_all = jnp.concatenate([edge_index_u2i[1], edge_index_i2u[0]])
    key = u_all.astype(jnp.uint32) * jnp.uint32(_NI) + i_all.astype(jnp.uint32)

    @compute_on("tpu_sparsecore")
    @jax.jit
    def _sc_sort(k):
        return lax.sort(k)

    skey = _sc_sort(key)
    first = jnp.concatenate(
        [jnp.ones((1,), bool), skey[1:] != skey[:-1]]
    )
    su = (skey // jnp.uint32(_NI)).astype(jnp.int32)
    si = (skey % jnp.uint32(_NI)).astype(jnp.int32)

    e_tot = su.shape[0]
    dummy = (jnp.arange(e_tot, dtype=jnp.int32) % _NPAD) + _NU
    su_eff = jnp.where(first, su, dummy)
    si_eff = jnp.where(first, si, dummy)
    pad_idx = (jnp.arange(_EPAD - e_tot, dtype=jnp.int32) % _NPAD) + _NU
    su_eff = jnp.concatenate([su_eff, pad_idx])
    si_eff = jnp.concatenate([si_eff, pad_idx])
    zeros_tab = jnp.zeros((_NT, _H), jnp.float32)
    zpad = jnp.zeros((_NPAD, _H), jnp.float32)

    du, di = _deg_dinv_sc(su_eff, si_eff, jnp.zeros((_NT,), jnp.float32))
    dinv_u = jnp.where(du[:_NU] > 0, du[:_NU] ** -0.5, 0.0)
    dinv_i = jnp.where(di[:_NU] > 0, di[:_NU] ** -0.5, 0.0)

    alpha = 1.0 / (_DEPTH + 1)
    xu, xi = user_emb, item_emb
    ou = xu * alpha
    oi = xi * alpha
    for _ in range(_DEPTH):
        xsu = jnp.concatenate([xu * dinv_u[:, None], zpad])
        xsi = jnp.concatenate([xi * dinv_i[:, None], zpad])
        au, ai = _propagate_sc(xsu, xsi, su_eff, si_eff, zeros_tab)
        xu = au * dinv_u[:, None]
        xi = ai * dinv_i[:, None]
        ou = ou + xu * alpha
        oi = oi + xi * alpha

    return _decoder_sc(ou, oi, edge_label_index[0], edge_label_index[1])


# speedup vs baseline: 27.3165x; 1.1072x over previous
import functools

import jax
import jax.numpy as jnp
from jax import lax
from jax.experimental import pallas as pl
from jax.experimental.pallas import tpu as pltpu
from jax.experimental.pallas import tpu_sc as plsc
from jax.experimental.compute_on import compute_on

_NU = 50000
_NI = 50000
_H = 32
_DEPTH = 2
_B = 16384

_info = plsc.get_sparse_core_info()
_NC, _NS = _info.num_cores, _info.num_subcores
_NW = _NC * _NS


def _decoder_sc(out_user, out_item, idx_u, idx_i):
    b_per_w = _B // _NW
    mesh = plsc.VectorSubcoreMesh(core_axis_name="c", subcore_axis_name="s")

    @functools.partial(
        pl.kernel,
        mesh=mesh,
        compiler_params=pltpu.CompilerParams(
            needs_layout_passes=False, use_tc_tiling_on_sc=False
        ),
        out_type=jax.ShapeDtypeStruct((_B,), jnp.float32),
        scratch_types=[
            pltpu.VMEM((b_per_w,), jnp.int32),
            pltpu.VMEM((b_per_w,), jnp.int32),
            pltpu.VMEM((b_per_w, _H), jnp.float32),
            pltpu.VMEM((b_per_w, _H), jnp.float32),
            pltpu.VMEM((b_per_w,), jnp.float32),
            pltpu.SemaphoreType.DMA,
        ],
    )
    def dec(u_hbm, i_hbm, iu_hbm, ii_hbm, out_hbm, iu_v, ii_v, ru_v, ri_v, o_v, sem):
        wid = lax.axis_index("s") * _NC + lax.axis_index("c")
        base = wid * b_per_w
        pltpu.sync_copy(iu_hbm.at[pl.ds(base, b_per_w)], iu_v)
        pltpu.sync_copy(ii_hbm.at[pl.ds(base, b_per_w)], ii_v)
        pltpu.async_copy(u_hbm.at[iu_v], ru_v, sem).wait()
        pltpu.async_copy(i_hbm.at[ii_v], ri_v, sem).wait()

        def body(g, carry):
            lanes = lax.iota(jnp.int32, 16)
            dots = jnp.zeros((16,), jnp.float32)
            for l in range(16):
                j = g * 16 + l
                a = ru_v[j, pl.ds(0, 16)] * ri_v[j, pl.ds(0, 16)]
                b = ru_v[j, pl.ds(16, 16)] * ri_v[j, pl.ds(16, 16)]
                s = jnp.sum(a + b)
                dots = jnp.where(lanes == l, s, dots)
            o_v[pl.ds(g * 16, 16)] = dots
            return carry

        lax.fori_loop(0, b_per_w // 16, body, 0)
        pltpu.sync_copy(o_v, out_hbm.at[pl.ds(base, b_per_w)])

    return dec(out_user, out_item, idx_u, idx_i)


_NPAD = 48
_NT = _NU + _NPAD
_CH = 400
_EPT = 102400
_EPAD = _EPT * _NS


def _deg_dinv_sc(su_eff, si_eff, zeros_vec):
    mesh = plsc.VectorSubcoreMesh(core_axis_name="c", subcore_axis_name="s")
    rpt = _NT // _NS

    @functools.partial(
        pl.kernel,
        mesh=mesh,
        compiler_params=pltpu.CompilerParams(
            needs_layout_passes=False, use_tc_tiling_on_sc=False
        ),
        out_type=(
            jax.ShapeDtypeStruct((_NT,), jnp.float32),
            jax.ShapeDtypeStruct((_NT,), jnp.float32),
        ),
        scratch_types=[
            pltpu.VMEM((_CH,), jnp.int32),
            pltpu.VMEM((_CH,), jnp.float32),
            pltpu.VMEM_SHARED((_NT,), jnp.float32),
        ],
    )
    def degk(su_hbm, si_hbm, z_hbm, du_hbm, di_hbm, idx_v, ones_v, deg_sh):
        c = lax.axis_index("c")
        t = lax.axis_index("s")
        pltpu.sync_copy(z_hbm.at[pl.ds(t * rpt, rpt)], deg_sh.at[pl.ds(t * rpt, rpt)])
        for j in range(_CH // 16):
            ones_v[pl.ds(16 * j, 16)] = jnp.full((16,), 1.0, jnp.float32)
        plsc.subcore_barrier()

        def run(sidx_hbm):
            def body(g, carry):
                e0 = t * _EPT + g * _CH
                pltpu.sync_copy(sidx_hbm.at[pl.ds(e0, _CH)], idx_v)
                pltpu.sync_copy(ones_v, deg_sh.at[idx_v], add=True)
                return carry

            lax.fori_loop(0, _EPT // _CH, body, 0)

        @pl.when(c == 0)
        def _():
            run(su_hbm)

        @pl.when(c == 1)
        def _():
            run(si_hbm)

        plsc.subcore_barrier()

        @pl.when(c == 0)
        def _():
            pltpu.sync_copy(deg_sh.at[pl.ds(t * rpt, rpt)], du_hbm.at[pl.ds(t * rpt, rpt)])

        @pl.when(c == 1)
        def _():
            pltpu.sync_copy(deg_sh.at[pl.ds(t * rpt, rpt)], di_hbm.at[pl.ds(t * rpt, rpt)])

    return degk(su_eff, si_eff, zeros_vec)


def _propagate_sc(xsu, xsi, su_eff, si_eff, zeros_tab):
    mesh = plsc.VectorSubcoreMesh(core_axis_name="c", subcore_axis_name="s")

    @functools.partial(
        pl.kernel,
        mesh=mesh,
        compiler_params=pltpu.CompilerParams(
            needs_layout_passes=False, use_tc_tiling_on_sc=False
        ),
        out_type=(
            jax.ShapeDtypeStruct((_NU, _H), jnp.float32),
            jax.ShapeDtypeStruct((_NI, _H), jnp.float32),
        ),
        scratch_types=[
            pltpu.VMEM((2, _CH), jnp.int32),
            pltpu.VMEM((2, _CH), jnp.int32),
            pltpu.VMEM((2, _CH, _H), jnp.float32),
            pltpu.VMEM_SHARED((_NT, _H), jnp.float32),
            pltpu.SemaphoreType.DMA((2,)),
            pltpu.SemaphoreType.DMA((2,)),
            pltpu.SemaphoreType.DMA((2,)),
        ],
    )
    def prop(xu_hbm, xi_hbm, su_hbm, si_hbm, z_hbm, ou_hbm, oi_hbm,
             gi_v, si_v, rows_v, acc_sh, isem, gsem, ssem):
        c = lax.axis_index("c")
        t = lax.axis_index("s")
        rpt = _NT // _NS
        pltpu.sync_copy(z_hbm.at[pl.ds(t * rpt, rpt)], acc_sh.at[pl.ds(t * rpt, rpt)])
        plsc.subcore_barrier()

        nchunk = _EPT // _CH

        def run(gtab_hbm, gidx_hbm, sidx_hbm):
            def start_idx(g, b):
                e0 = t * _EPT + g * _CH
                pltpu.async_copy(gidx_hbm.at[pl.ds(e0, _CH)], gi_v.at[b], isem.at[b])
                pltpu.async_copy(sidx_hbm.at[pl.ds(e0, _CH)], si_v.at[b], isem.at[b])

            def wait_idx(b):
                pltpu.make_async_copy(gidx_hbm.at[pl.ds(0, _CH)], gi_v.at[b], isem.at[b]).wait()
                pltpu.make_async_copy(sidx_hbm.at[pl.ds(0, _CH)], si_v.at[b], isem.at[b]).wait()

            def wait_rows(b, sem):
                pltpu.make_async_copy(
                    gtab_hbm.at[pl.ds(0, _CH)], rows_v.at[b], sem.at[b]
                ).wait()

            start_idx(0, 0)

            def outer(g0, carry):
                for b in range(2):
                    g = g0 * 2 + b
                    o = 1 - b
                    wait_idx(b)
                    pltpu.async_copy(gtab_hbm.at[gi_v.at[b]], rows_v.at[b], gsem.at[b])

                    @pl.when(g > 0)
                    def _():
                        wait_rows(o, ssem)

                    @pl.when(g + 1 < nchunk)
                    def _():
                        start_idx(g + 1, o)

                    wait_rows(b, gsem)
                    pltpu.async_copy(
                        rows_v.at[b], acc_sh.at[si_v.at[b]], ssem.at[b], add=True
                    )
                return carry

            lax.fori_loop(0, nchunk // 2, outer, 0)
            wait_rows(1, ssem)

        @pl.when(c == 0)
        def _():
            run(xi_hbm, si_hbm, su_hbm)

        @pl.when(c == 1)
        def _():
            run(xu_hbm, su_hbm, si_hbm)

        plsc.subcore_barrier()
        wpt = _NU // _NS

        @pl.when(c == 0)
        def _():
            pltpu.sync_copy(
                acc_sh.at[pl.ds(t * wpt, wpt)], ou_hbm.at[pl.ds(t * wpt, wpt)]
            )

        @pl.when(c == 1)
        def _():
            pltpu.sync_copy(
                acc_sh.at[pl.ds(t * wpt, wpt)], oi_hbm.at[pl.ds(t * wpt, wpt)]
            )

    return prop(xsu, xsi, su_eff, si_eff, zeros_tab)


def kernel(user_emb, item_emb, edge_index_u2i, edge_index_i2u, edge_label_index):
    u_all = jnp.concatenate([edge_index_u2i[0], edge_index_i2u[1]])
    i_all = jnp.concatenate([edge_index_u2i[1], edge_index_i2u[0]])
    key = u_all.astype(jnp.uint32) * jnp.uint32(_NI) + i_all.astype(jnp.uint32)

    @compute_on("tpu_sparsecore")
    @jax.jit
    def _sc_sort(k):
        return lax.sort(k)

    skey = _sc_sort(key)
    first = jnp.concatenate(
        [jnp.ones((1,), bool), skey[1:] != skey[:-1]]
    )
    su = (skey // jnp.uint32(_NI)).astype(jnp.int32)
    si = (skey % jnp.uint32(_NI)).astype(jnp.int32)

    e_tot = su.shape[0]
    dummy = (jnp.arange(e_tot, dtype=jnp.int32) % _NPAD) + _NU
    su_eff = jnp.where(first, su, dummy)
    si_eff = jnp.where(first, si, dummy)
    pad_idx = (jnp.arange(_EPAD - e_tot, dtype=jnp.int32) % _NPAD) + _NU
    su_eff = jnp.concatenate([su_eff, pad_idx])
    si_eff = jnp.concatenate([si_eff, pad_idx])
    zeros_tab = jnp.zeros((_NT, _H), jnp.float32)
    zpad = jnp.zeros((_NPAD, _H), jnp.float32)

    du, di = _deg_dinv_sc(su_eff, si_eff, jnp.zeros((_NT,), jnp.float32))
    dinv_u = jnp.where(du[:_NU] > 0, du[:_NU] ** -0.5, 0.0)
    dinv_i = jnp.where(di[:_NU] > 0, di[:_NU] ** -0.5, 0.0)

    alpha = 1.0 / (_DEPTH + 1)
    xu, xi = user_emb, item_emb
    ou = xu * alpha
    oi = xi * alpha
    for _ in range(_DEPTH):
        xsu = jnp.concatenate([xu * dinv_u[:, None], zpad])
        xsi = jnp.concatenate([xi * dinv_i[:, None], zpad])
        au, ai = _propagate_sc(xsu, xsi, su_eff, si_eff, zeros_tab)
        xu = au * dinv_u[:, None]
        xi = ai * dinv_i[:, None]
        ou = ou + xu * alpha
        oi = oi + xi * alpha

    return _decoder_sc(ou, oi, edge_label_index[0], edge_label_index[1])


# speedup vs baseline: 28.0355x; 1.0263x over previous
import functools

import jax
import jax.numpy as jnp
from jax import lax
from jax.experimental import pallas as pl
from jax.experimental.pallas import tpu as pltpu
from jax.experimental.pallas import tpu_sc as plsc
from jax.experimental.compute_on import compute_on

_NU = 50000
_NI = 50000
_H = 32
_DEPTH = 2
_B = 16384

_info = plsc.get_sparse_core_info()
_NC, _NS = _info.num_cores, _info.num_subcores
_NW = _NC * _NS


def _decoder_sc(out_user, out_item, idx_u, idx_i):
    b_per_w = _B // _NW
    mesh = plsc.VectorSubcoreMesh(core_axis_name="c", subcore_axis_name="s")

    @functools.partial(
        pl.kernel,
        mesh=mesh,
        compiler_params=pltpu.CompilerParams(
            needs_layout_passes=False, use_tc_tiling_on_sc=False
        ),
        out_type=jax.ShapeDtypeStruct((_B,), jnp.float32),
        scratch_types=[
            pltpu.VMEM((b_per_w,), jnp.int32),
            pltpu.VMEM((b_per_w,), jnp.int32),
            pltpu.VMEM((b_per_w, _H), jnp.float32),
            pltpu.VMEM((b_per_w, _H), jnp.float32),
            pltpu.VMEM((b_per_w,), jnp.float32),
            pltpu.SemaphoreType.DMA,
        ],
    )
    def dec(u_hbm, i_hbm, iu_hbm, ii_hbm, out_hbm, iu_v, ii_v, ru_v, ri_v, o_v, sem):
        wid = lax.axis_index("s") * _NC + lax.axis_index("c")
        base = wid * b_per_w
        pltpu.sync_copy(iu_hbm.at[pl.ds(base, b_per_w)], iu_v)
        pltpu.sync_copy(ii_hbm.at[pl.ds(base, b_per_w)], ii_v)
        pltpu.async_copy(u_hbm.at[iu_v], ru_v, sem).wait()
        pltpu.async_copy(i_hbm.at[ii_v], ri_v, sem).wait()

        def body(g, carry):
            lanes = lax.iota(jnp.int32, 16)
            dots = jnp.zeros((16,), jnp.float32)
            for l in range(16):
                j = g * 16 + l
                a = ru_v[j, pl.ds(0, 16)] * ri_v[j, pl.ds(0, 16)]
                b = ru_v[j, pl.ds(16, 16)] * ri_v[j, pl.ds(16, 16)]
                s = jnp.sum(a + b)
                dots = jnp.where(lanes == l, s, dots)
            o_v[pl.ds(g * 16, 16)] = dots
            return carry

        lax.fori_loop(0, b_per_w // 16, body, 0)
        pltpu.sync_copy(o_v, out_hbm.at[pl.ds(base, b_per_w)])

    return dec(out_user, out_item, idx_u, idx_i)


_NPAD = 48
_NT = _NU + _NPAD
_CH = 400
_CHD = 6400
_EPT = 102400
_EPAD = _EPT * _NS


def _deg_dinv_sc(su_eff, si_eff, zeros_vec):
    mesh = plsc.VectorSubcoreMesh(core_axis_name="c", subcore_axis_name="s")
    rpt = _NT // _NS

    @functools.partial(
        pl.kernel,
        mesh=mesh,
        compiler_params=pltpu.CompilerParams(
            needs_layout_passes=False, use_tc_tiling_on_sc=False
        ),
        out_type=(
            jax.ShapeDtypeStruct((_NT,), jnp.float32),
            jax.ShapeDtypeStruct((_NT,), jnp.float32),
        ),
        scratch_types=[
            pltpu.VMEM((_CHD,), jnp.int32),
            pltpu.VMEM((_CHD,), jnp.float32),
            pltpu.VMEM_SHARED((_NT,), jnp.float32),
        ],
    )
    def degk(su_hbm, si_hbm, z_hbm, du_hbm, di_hbm, idx_v, ones_v, deg_sh):
        c = lax.axis_index("c")
        t = lax.axis_index("s")
        pltpu.sync_copy(z_hbm.at[pl.ds(t * rpt, rpt)], deg_sh.at[pl.ds(t * rpt, rpt)])

        def fill(j, carry):
            ones_v[pl.ds(16 * j, 16)] = jnp.full((16,), 1.0, jnp.float32)
            return carry

        lax.fori_loop(0, _CHD // 16, fill, 0)
        plsc.subcore_barrier()

        def run(sidx_hbm):
            def body(g, carry):
                e0 = t * _EPT + g * _CHD
                pltpu.sync_copy(sidx_hbm.at[pl.ds(e0, _CHD)], idx_v)
                pltpu.sync_copy(ones_v, deg_sh.at[idx_v], add=True)
                return carry

            lax.fori_loop(0, _EPT // _CHD, body, 0)

        @pl.when(c == 0)
        def _():
            run(su_hbm)

        @pl.when(c == 1)
        def _():
            run(si_hbm)

        plsc.subcore_barrier()

        @pl.when(c == 0)
        def _():
            pltpu.sync_copy(deg_sh.at[pl.ds(t * rpt, rpt)], du_hbm.at[pl.ds(t * rpt, rpt)])

        @pl.when(c == 1)
        def _():
            pltpu.sync_copy(deg_sh.at[pl.ds(t * rpt, rpt)], di_hbm.at[pl.ds(t * rpt, rpt)])

    return degk(su_eff, si_eff, zeros_vec)


def _propagate_sc(xsu, xsi, su_eff, si_eff, zeros_tab):
    mesh = plsc.VectorSubcoreMesh(core_axis_name="c", subcore_axis_name="s")

    @functools.partial(
        pl.kernel,
        mesh=mesh,
        compiler_params=pltpu.CompilerParams(
            needs_layout_passes=False, use_tc_tiling_on_sc=False
        ),
        out_type=(
            jax.ShapeDtypeStruct((_NU, _H), jnp.float32),
            jax.ShapeDtypeStruct((_NI, _H), jnp.float32),
        ),
        scratch_types=[
            pltpu.VMEM((2, _CH), jnp.int32),
            pltpu.VMEM((2, _CH), jnp.int32),
            pltpu.VMEM((2, _CH, _H), jnp.float32),
            pltpu.VMEM_SHARED((_NT, _H), jnp.float32),
            pltpu.SemaphoreType.DMA((2,)),
            pltpu.SemaphoreType.DMA((2,)),
            pltpu.SemaphoreType.DMA((2,)),
        ],
    )
    def prop(xu_hbm, xi_hbm, su_hbm, si_hbm, z_hbm, ou_hbm, oi_hbm,
             gi_v, si_v, rows_v, acc_sh, isem, gsem, ssem):
        c = lax.axis_index("c")
        t = lax.axis_index("s")
        rpt = _NT // _NS
        pltpu.sync_copy(z_hbm.at[pl.ds(t * rpt, rpt)], acc_sh.at[pl.ds(t * rpt, rpt)])
        plsc.subcore_barrier()

        nchunk = _EPT // _CH

        def run(gtab_hbm, gidx_hbm, sidx_hbm):
            def start_idx(g, b):
                e0 = t * _EPT + g * _CH
                pltpu.async_copy(gidx_hbm.at[pl.ds(e0, _CH)], gi_v.at[b], isem.at[b])
                pltpu.async_copy(sidx_hbm.at[pl.ds(e0, _CH)], si_v.at[b], isem.at[b])

            def wait_idx(b):
                pltpu.make_async_copy(gidx_hbm.at[pl.ds(0, _CH)], gi_v.at[b], isem.at[b]).wait()
                pltpu.make_async_copy(sidx_hbm.at[pl.ds(0, _CH)], si_v.at[b], isem.at[b]).wait()

            def wait_rows(b, sem):
                pltpu.make_async_copy(
                    gtab_hbm.at[pl.ds(0, _CH)], rows_v.at[b], sem.at[b]
                ).wait()

            start_idx(0, 0)

            def outer(g0, carry):
                for b in range(2):
                    g = g0 * 2 + b
                    o = 1 - b
                    wait_idx(b)
                    pltpu.async_copy(gtab_hbm.at[gi_v.at[b]], rows_v.at[b], gsem.at[b])

                    @pl.when(g > 0)
                    def _():
                        wait_rows(o, ssem)

                    @pl.when(g + 1 < nchunk)
                    def _():
                        start_idx(g + 1, o)

                    wait_rows(b, gsem)
                    pltpu.async_copy(
                        rows_v.at[b], acc_sh.at[si_v.at[b]], ssem.at[b], add=True
                    )
                return carry

            lax.fori_loop(0, nchunk // 2, outer, 0)
            wait_rows(1, ssem)

        @pl.when(c == 0)
        def _():
            run(xi_hbm, si_hbm, su_hbm)

        @pl.when(c == 1)
        def _():
            run(xu_hbm, su_hbm, si_hbm)

        plsc.subcore_barrier()
        wpt = _NU // _NS

        @pl.when(c == 0)
        def _():
            pltpu.sync_copy(
                acc_sh.at[pl.ds(t * wpt, wpt)], ou_hbm.at[pl.ds(t * wpt, wpt)]
            )

        @pl.when(c == 1)
        def _():
            pltpu.sync_copy(
                acc_sh.at[pl.ds(t * wpt, wpt)], oi_hbm.at[pl.ds(t * wpt, wpt)]
            )

    return prop(xsu, xsi, su_eff, si_eff, zeros_tab)


def kernel(user_emb, item_emb, edge_index_u2i, edge_index_i2u, edge_label_index):
    u_all = jnp.concatenate([edge_index_u2i[0], edge_index_i2u[1]])
    i_all = jnp.concatenate([edge_index_u2i[1], edge_index_i2u[0]])
    key = u_all.astype(jnp.uint32) * jnp.uint32(_NI) + i_all.astype(jnp.uint32)

    @compute_on("tpu_sparsecore")
    @jax.jit
    def _sc_sort(k):
        return lax.sort(k)

    skey = _sc_sort(key)
    first = jnp.concatenate(
        [jnp.ones((1,), bool), skey[1:] != skey[:-1]]
    )
    su = (skey // jnp.uint32(_NI)).astype(jnp.int32)
    si = (skey % jnp.uint32(_NI)).astype(jnp.int32)

    e_tot = su.shape[0]
    dummy = (jnp.arange(e_tot, dtype=jnp.int32) % _NPAD) + _NU
    su_eff = jnp.where(first, su, dummy)
    si_eff = jnp.where(first, si, dummy)
    pad_idx = (jnp.arange(_EPAD - e_tot, dtype=jnp.int32) % _NPAD) + _NU
    su_eff = jnp.concatenate([su_eff, pad_idx])
    si_eff = jnp.concatenate([si_eff, pad_idx])
    zeros_tab = jnp.zeros((_NT, _H), jnp.float32)
    zpad = jnp.zeros((_NPAD, _H), jnp.float32)

    du, di = _deg_dinv_sc(su_eff, si_eff, jnp.zeros((_NT,), jnp.float32))
    dinv_u = jnp.where(du[:_NU] > 0, du[:_NU] ** -0.5, 0.0)
    dinv_i = jnp.where(di[:_NU] > 0, di[:_NU] ** -0.5, 0.0)

    alpha = 1.0 / (_DEPTH + 1)
    xu, xi = user_emb, item_emb
    ou = xu * alpha
    oi = xi * alpha
    for _ in range(_DEPTH):
        xsu = jnp.concatenate([xu * dinv_u[:, None], zpad])
        xsi = jnp.concatenate([xi * dinv_i[:, None], zpad])
        au, ai = _propagate_sc(xsu, xsi, su_eff, si_eff, zeros_tab)
        xu = au * dinv_u[:, None]
        xi = ai * dinv_i[:, None]
        ou = ou + xu * alpha
        oi = oi + xi * alpha

    return _decoder_sc(ou, oi, edge_label_index[0], edge_label_index[1])
